# baseline (speedup 1.0000x reference)
import functools

import jax
import jax.numpy as jnp
from jax import lax
from jax.experimental import pallas as pl
from jax.experimental.pallas import tpu as pltpu
from jax.experimental.pallas import tpu_sc as plsc

N_NODES = 100000
MAX_DEG = 32
D = 128
H = 32
N1 = 25
N2 = 10
B0 = 1024
B1 = B0 * N1
NC = 2
NS = 16
NW = NC * NS
L = 16
PW = B1 // NW
CH = PW // 4
SEEDS_PW = B0 // NW

@functools.cache
def _make_sc_gather():
  mesh = plsc.VectorSubcoreMesh(core_axis_name="c", subcore_axis_name="s")
  return functools.partial(
    pl.kernel,
    mesh=mesh,
    compiler_params=pltpu.CompilerParams(
        needs_layout_passes=False, use_tc_tiling_on_sc=False),
    out_type=[
        jax.ShapeDtypeStruct((B0, D), jnp.float32),
        jax.ShapeDtypeStruct((B1, D), jnp.float32),
        jax.ShapeDtypeStruct((N2, B1, D), jnp.float32),
    ],
    scratch_types=[
        pltpu.VMEM((B0,), jnp.int32),
        pltpu.VMEM((32,), jnp.int32),
        pltpu.VMEM((N2, L), jnp.int32),
        pltpu.VMEM((PW,), jnp.int32),
        pltpu.VMEM((PW,), jnp.int32),
        pltpu.VMEM((PW, MAX_DEG), jnp.int32),
        pltpu.VMEM((PW,), jnp.int32),
        pltpu.VMEM((PW,), jnp.int32),
        pltpu.VMEM((PW,), jnp.int32),
        pltpu.VMEM((CH, D), jnp.float32),
        pltpu.VMEM((CH, D), jnp.float32),
        pltpu.SemaphoreType.DMA,
        pltpu.SemaphoreType.DMA,
        pltpu.SemaphoreType.DMA,
        pltpu.SemaphoreType.DMA,
        pltpu.SemaphoreType.DMA,
    ],
  )(_sc_gather_body)


def _sc_gather_body(ids_hbm, adj_hbm, feats_hbm, cols1_hbm, cols2_hbm,
               f0_hbm, f1_hbm, f2_hbm,
               ids_v, cols1_v, cols2_v, rowids_v, colsel_v, adjrows_v,
               ids1p_v, idx2a_v, idx2b_v, bufa, bufb,
               sema, semb, sem_adj, semwa, semwb):
    wid = lax.axis_index("s") * NC + lax.axis_index("c")
    p0 = wid * PW
    bufs, sems, semws = (bufa, bufb), (sema, semb), (semwa, semwb)
    idxbufs = (idx2a_v, idx2b_v)

    pltpu.sync_copy(ids_hbm, ids_v)
    pltpu.sync_copy(cols1_hbm, cols1_v)
    pltpu.sync_copy(cols2_hbm, cols2_v)

    def idx_body(c, carry):
        off = pl.multiple_of(c * L, L)
        p = p0 + off + lax.iota(jnp.int32, L)
        i = lax.shift_right_logical(p, 10)
        b = lax.bitwise_and(p, B0 - 1)
        rowids_v[pl.ds(off, L)] = plsc.load_gather(ids_v, [b])
        colsel_v[pl.ds(off, L)] = plsc.load_gather(cols1_v, [i])
        return carry
    lax.fori_loop(0, PW // L, idx_body, 0)

    pltpu.async_copy(adj_hbm.at[rowids_v], adjrows_v, sem_adj).wait()

    def sel_body(c, carry):
        off = pl.multiple_of(c * L, L)
        lidx = off + lax.iota(jnp.int32, L)
        cs = colsel_v[pl.ds(off, L)]
        ids1p_v[pl.ds(off, L)] = plsc.load_gather(adjrows_v, [lidx, cs])
        return carry
    lax.fori_loop(0, PW // L, sel_body, 0)

    adj2 = pltpu.async_copy(adj_hbm.at[ids1p_v], adjrows_v, sem_adj)

    def _sel2(jcol, dst):
        def body(c, carry):
            off = pl.multiple_of(c * L, L)
            lidx = off + lax.iota(jnp.int32, L)
            dst[pl.ds(off, L)] = plsc.load_gather(adjrows_v, [lidx, jcol])
            return carry
        lax.fori_loop(0, PW // L, body, 0)

    NCH = PW // CH

    chunks = []
    for h in range(NCH):
        chunks.append((ids1p_v.at[pl.ds(h * CH, CH)],
                       f1_hbm.at[pl.ds(p0 + h * CH, CH)],
                       True if h == 0 else None))
    for j in range(N2):
        for h in range(NCH):
            chunks.append((idxbufs[j & 1].at[pl.ds(h * CH, CH)],
                           f2_hbm.at[j, pl.ds(p0 + h * CH, CH)],
                           (j + 1, h) if (h == 0 and j + 1 < N2) else None))

    prev = None
    wbd = [None, None]
    for n, (idx_slice, dst, hook) in enumerate(chunks):
        bi = n & 1
        if wbd[bi] is not None:
            wbd[bi].wait()
            wbd[bi] = None
        d = pltpu.async_copy(feats_hbm.at[idx_slice], bufs[bi], sems[bi])
        if prev is not None:
            pd, pbi, pdst = prev
            pd.wait()
            wbd[pbi] = pltpu.async_copy(bufs[pbi], pdst, semws[pbi])
        if hook is True:
            adj2.wait()
            _sel2(cols2_v[0, :], idxbufs[0])
        elif hook is not None:
            jn = hook[0]
            _sel2(cols2_v[jn, :], idxbufs[jn & 1])
        prev = (d, bi, dst)

    r0 = wid * SEEDS_PW
    pd, pbi, pdst = prev
    pd.wait()
    wbd[pbi] = pltpu.async_copy(bufs[pbi], pdst, semws[pbi])
    bi = 1 - pbi
    if wbd[bi] is not None:
        wbd[bi].wait()
        wbd[bi] = None
    d0 = pltpu.async_copy(
        feats_hbm.at[ids_v.at[pl.ds(r0, SEEDS_PW)]],
        bufs[bi].at[pl.ds(0, SEEDS_PW)], sems[bi])
    d0.wait()
    pltpu.sync_copy(bufs[bi].at[pl.ds(0, SEEDS_PW)],
                    f0_hbm.at[pl.ds(r0, SEEDS_PW)])
    wbd[pbi].wait()


def _softmax_weighted_sum(get_nb, n, xa, aw):
    scores = []
    for j in range(n):
        na = jnp.dot(get_nb(j), aw, preferred_element_type=jnp.float32)
        scores.append(jnp.sum(na * xa, axis=1, keepdims=True))
    m = scores[0]
    for s in scores[1:]:
        m = jnp.maximum(m, s)
    es = [jnp.exp(s - m) for s in scores]
    z = es[0]
    for e in es[1:]:
        z = z + e
    inv = 1.0 / z
    agg = get_nb(0) * (es[0] * inv)
    for j in range(1, n):
        agg = agg + get_nb(j) * (es[j] * inv)
    return agg


def _l1_body(f1_ref, f2_ref, aw_ref, xw_ref, nw_ref, out_ref):
    x = f1_ref[...]
    aw = aw_ref[...]
    xa = jnp.dot(x, aw, preferred_element_type=jnp.float32)
    f2all = f2_ref[...].reshape(N2 * B0, D)
    na = jnp.dot(f2all, aw, preferred_element_type=jnp.float32)
    xa_rep = jnp.concatenate([xa] * N2, axis=0)
    s_all = jnp.sum(na * xa_rep, axis=1, keepdims=True)
    scores = [lax.slice(s_all, (j * B0, 0), ((j + 1) * B0, 1)) for j in range(N2)]
    m = scores[0]
    for s in scores[1:]:
        m = jnp.maximum(m, s)
    es = [jnp.exp(s - m) for s in scores]
    z = es[0]
    for e in es[1:]:
        z = z + e
    inv = 1.0 / z
    agg = f2_ref[0] * (es[0] * inv)
    for j in range(1, N2):
        agg = agg + f2_ref[j] * (es[j] * inv)
    out_ref[:, :D] = jnp.maximum(
        jnp.dot(x, xw_ref[...], preferred_element_type=jnp.float32), 0.0)
    out_ref[:, D:] = jnp.maximum(
        jnp.dot(agg, nw_ref[...], preferred_element_type=jnp.float32), 0.0)


def _final_body(f0_ref, f1_ref, g1_ref, aw1_ref, xw1_ref, nw1_ref,
                aw2_ref, xw2_ref, nw2_ref, fcw_ref, fcb_ref, out_ref):
    x0 = f0_ref[...]
    aw1 = aw1_ref[...]
    xa = jnp.dot(x0, aw1, preferred_element_type=jnp.float32)
    agg0 = _softmax_weighted_sum(
        lambda i: f1_ref[pl.ds(i * B0, B0), :], N1, xa, aw1)
    g0 = jnp.concatenate([
        jnp.maximum(jnp.dot(x0, xw1_ref[...],
                            preferred_element_type=jnp.float32), 0.0),
        jnp.maximum(jnp.dot(agg0, nw1_ref[...],
                            preferred_element_type=jnp.float32), 0.0)], axis=1)
    aw2 = aw2_ref[...]
    xa2 = jnp.dot(g0, aw2, preferred_element_type=jnp.float32)
    agg2 = _softmax_weighted_sum(
        lambda i: g1_ref[pl.ds(i * B0, B0), :], N1, xa2, aw2)
    h0 = jnp.concatenate([
        jnp.maximum(jnp.dot(g0, xw2_ref[...],
                            preferred_element_type=jnp.float32), 0.0),
        jnp.maximum(jnp.dot(agg2, nw2_ref[...],
                            preferred_element_type=jnp.float32), 0.0)], axis=1)
    nrm = jnp.sqrt(jnp.sum(h0 * h0, axis=1, keepdims=True))
    hn = h0 / jnp.maximum(nrm, 1e-12)
    out_ref[...] = (jnp.dot(hn, fcw_ref[...],
                            preferred_element_type=jnp.float32)
                    + fcb_ref[...])


def _run_final(f0, f1, g1, att_w1, fcx_w1, fcn_w1, att_w2, fcx_w2, fcn_w2,
               fc_w, fc_b):
    return pl.pallas_call(
        _final_body,
        compiler_params=pltpu.CompilerParams(vmem_limit_bytes=100 * 2**20),
        out_shape=jax.ShapeDtypeStruct((B0, fc_w.shape[1]), jnp.float32),
    )(f0, f1, g1, att_w1, fcx_w1, fcn_w1, att_w2, fcx_w2, fcn_w2,
      fc_w, fc_b.reshape(1, -1))


def kernel(ids, feats, adj, att_w1, fcx_w1, fcn_w1, att_w2, fcx_w2, fcn_w2,
           fc_w, fc_b):
    k = jax.random.key(42)
    perm1 = jax.random.permutation(jax.random.fold_in(k, 0), MAX_DEG)
    perm2 = jax.random.permutation(jax.random.fold_in(k, 1), MAX_DEG)
    cols1 = jnp.zeros((32,), jnp.int32).at[:N1].set(perm1[:N1].astype(jnp.int32))
    cols2 = jnp.broadcast_to(
        perm2[:N2].astype(jnp.int32)[:, None], (N2, L))

    f0, f1, f2 = _make_sc_gather()(ids.astype(jnp.int32), adj.astype(jnp.int32),
                                   feats, cols1, cols2)

    g1 = pl.pallas_call(
        _l1_body,
        grid=(N1,),
        in_specs=[
            pl.BlockSpec((B0, D), lambda kk: (kk, 0)),
            pl.BlockSpec((N2, B0, D), lambda kk: (0, kk, 0)),
            pl.BlockSpec((D, H), lambda kk: (0, 0)),
            pl.BlockSpec((D, D), lambda kk: (0, 0)),
            pl.BlockSpec((D, D), lambda kk: (0, 0)),
        ],
        out_specs=pl.BlockSpec((B0, 2 * D), lambda kk: (kk, 0)),
        out_shape=jax.ShapeDtypeStruct((B1, 2 * D), jnp.float32),
    )(f1, f2, att_w1, fcx_w1, fcn_w1)

    return _run_final(f0, f1, g1, att_w1, fcx_w1, fcn_w1,
                      att_w2, fcx_w2, fcn_w2, fc_w, fc_b)


# speedup vs baseline: 1.0087x; 1.0087x over previous
import functools

import jax
import jax.numpy as jnp
from jax import lax
from jax.experimental import pallas as pl
from jax.experimental.pallas import tpu as pltpu
from jax.experimental.pallas import tpu_sc as plsc

N_NODES = 100000
MAX_DEG = 32
D = 128
H = 32
N1 = 25
N2 = 10
B0 = 1024
B1 = B0 * N1
NC = 2
NS = 16
NW = NC * NS
L = 16
PW = B1 // NW
CH = PW // 4
SEEDS_PW = B0 // NW
NB_A = 13
NB_B = N1 - NB_A

@functools.cache
def _make_sc_gather(p_base, rows, with_f0):
  mesh = plsc.VectorSubcoreMesh(core_axis_name="c", subcore_axis_name="s")
  pw = rows // NW
  ch = pw // 4
  assert pw % L == 0 and ch % 8 == 0 and ch * 4 == pw

  def body(ids_hbm, adj_hbm, feats_hbm, cols1_hbm, cols2_hbm, *rest):
    if with_f0:
        f0_hbm, f1_hbm, f2_hbm = rest[0], rest[1], rest[2]
        rest = rest[3:]
    else:
        f1_hbm, f2_hbm = rest[0], rest[1]
        rest = rest[2:]
    (ids_v, cols1_v, cols2_v, rowids_v, colsel_v, adjrows_v,
     ids1p_v, idx2a_v, idx2b_v, bufa, bufb,
     sema, semb, sem_adj, semwa, semwb) = rest
    wid = lax.axis_index("s") * NC + lax.axis_index("c")
    p0 = wid * pw
    pg = p_base + p0
    bufs, sems, semws = (bufa, bufb), (sema, semb), (semwa, semwb)
    idxbufs = (idx2a_v, idx2b_v)

    pltpu.sync_copy(ids_hbm, ids_v)
    pltpu.sync_copy(cols1_hbm, cols1_v)
    pltpu.sync_copy(cols2_hbm, cols2_v)

    def idx_body(c, carry):
        off = pl.multiple_of(c * L, L)
        p = pg + off + lax.iota(jnp.int32, L)
        i = lax.shift_right_logical(p, 10)
        b = lax.bitwise_and(p, B0 - 1)
        rowids_v[pl.ds(off, L)] = plsc.load_gather(ids_v, [b])
        colsel_v[pl.ds(off, L)] = plsc.load_gather(cols1_v, [i])
        return carry
    lax.fori_loop(0, pw // L, idx_body, 0)

    pltpu.async_copy(adj_hbm.at[rowids_v], adjrows_v, sem_adj).wait()

    def sel_body(c, carry):
        off = pl.multiple_of(c * L, L)
        lidx = off + lax.iota(jnp.int32, L)
        cs = colsel_v[pl.ds(off, L)]
        ids1p_v[pl.ds(off, L)] = plsc.load_gather(adjrows_v, [lidx, cs])
        return carry
    lax.fori_loop(0, pw // L, sel_body, 0)

    adj2 = pltpu.async_copy(adj_hbm.at[ids1p_v], adjrows_v, sem_adj)

    def _sel2(jcol, dst):
        def sbody(c, carry):
            off = pl.multiple_of(c * L, L)
            lidx = off + lax.iota(jnp.int32, L)
            dst[pl.ds(off, L)] = plsc.load_gather(adjrows_v, [lidx, jcol])
            return carry
        lax.fori_loop(0, pw // L, sbody, 0)

    chunks = []
    for h in range(4):
        chunks.append((ids1p_v.at[pl.ds(h * ch, ch)],
                       f1_hbm.at[pl.ds(p0 + h * ch, ch)],
                       True if h == 0 else None))
    for j in range(N2):
        for h in range(4):
            chunks.append((idxbufs[j & 1].at[pl.ds(h * ch, ch)],
                           f2_hbm.at[j, pl.ds(p0 + h * ch, ch)],
                           (j + 1, h) if (h == 0 and j + 1 < N2) else None))

    prev = None
    wbd = [None, None]
    for n, (idx_slice, dst, hook) in enumerate(chunks):
        bi = n & 1
        if wbd[bi] is not None:
            wbd[bi].wait()
            wbd[bi] = None
        d = pltpu.async_copy(feats_hbm.at[idx_slice], bufs[bi], sems[bi])
        if prev is not None:
            pd, pbi, pdst = prev
            pd.wait()
            wbd[pbi] = pltpu.async_copy(bufs[pbi], pdst, semws[pbi])
        if hook is True:
            adj2.wait()
            _sel2(cols2_v[0, :], idxbufs[0])
        elif hook is not None:
            jn = hook[0]
            _sel2(cols2_v[jn, :], idxbufs[jn & 1])
        prev = (d, bi, dst)

    pd, pbi, pdst = prev
    pd.wait()
    wbd[pbi] = pltpu.async_copy(bufs[pbi], pdst, semws[pbi])
    if with_f0:
        r0 = wid * SEEDS_PW
        bi = 1 - pbi
        if wbd[bi] is not None:
            wbd[bi].wait()
            wbd[bi] = None
        d0 = pltpu.async_copy(
            feats_hbm.at[ids_v.at[pl.ds(r0, SEEDS_PW)]],
            bufs[bi].at[pl.ds(0, SEEDS_PW)], sems[bi])
        d0.wait()
        pltpu.sync_copy(bufs[bi].at[pl.ds(0, SEEDS_PW)],
                        f0_hbm.at[pl.ds(r0, SEEDS_PW)])
    else:
        if wbd[1 - pbi] is not None:
            wbd[1 - pbi].wait()
    wbd[pbi].wait()

  out_type = []
  if with_f0:
      out_type.append(jax.ShapeDtypeStruct((B0, D), jnp.float32))
  out_type += [
      jax.ShapeDtypeStruct((rows, D), jnp.float32),
      jax.ShapeDtypeStruct((N2, rows, D), jnp.float32),
  ]
  return pl.kernel(
      body,
      mesh=mesh,
      compiler_params=pltpu.CompilerParams(
          needs_layout_passes=False, use_tc_tiling_on_sc=False),
      out_type=out_type,
      scratch_types=[
          pltpu.VMEM((B0,), jnp.int32),
          pltpu.VMEM((32,), jnp.int32),
          pltpu.VMEM((N2, L), jnp.int32),
          pltpu.VMEM((pw,), jnp.int32),
          pltpu.VMEM((pw,), jnp.int32),
          pltpu.VMEM((pw, MAX_DEG), jnp.int32),
          pltpu.VMEM((pw,), jnp.int32),
          pltpu.VMEM((pw,), jnp.int32),
          pltpu.VMEM((pw,), jnp.int32),
          pltpu.VMEM((ch, D), jnp.float32),
          pltpu.VMEM((ch, D), jnp.float32),
          pltpu.SemaphoreType.DMA,
          pltpu.SemaphoreType.DMA,
          pltpu.SemaphoreType.DMA,
          pltpu.SemaphoreType.DMA,
          pltpu.SemaphoreType.DMA,
      ],
  )


def _softmax_weighted_sum(get_nb, n, xa, aw):
    scores = []
    for j in range(n):
        na = jnp.dot(get_nb(j), aw, preferred_element_type=jnp.float32)
        scores.append(jnp.sum(na * xa, axis=1, keepdims=True))
    m = scores[0]
    for s in scores[1:]:
        m = jnp.maximum(m, s)
    es = [jnp.exp(s - m) for s in scores]
    z = es[0]
    for e in es[1:]:
        z = z + e
    inv = 1.0 / z
    agg = get_nb(0) * (es[0] * inv)
    for j in range(1, n):
        agg = agg + get_nb(j) * (es[j] * inv)
    return agg


def _l1_body(f1_ref, f2_ref, aw_ref, xw_ref, nw_ref, out_ref):
    x = f1_ref[...]
    aw = aw_ref[...]
    xa = jnp.dot(x, aw, preferred_element_type=jnp.float32)
    f2all = f2_ref[...].reshape(N2 * B0, D)
    na = jnp.dot(f2all, aw, preferred_element_type=jnp.float32)
    xa_rep = jnp.concatenate([xa] * N2, axis=0)
    s_all = jnp.sum(na * xa_rep, axis=1, keepdims=True)
    scores = [lax.slice(s_all, (j * B0, 0), ((j + 1) * B0, 1)) for j in range(N2)]
    m = scores[0]
    for s in scores[1:]:
        m = jnp.maximum(m, s)
    es = [jnp.exp(s - m) for s in scores]
    z = es[0]
    for e in es[1:]:
        z = z + e
    inv = 1.0 / z
    agg = f2_ref[0] * (es[0] * inv)
    for j in range(1, N2):
        agg = agg + f2_ref[j] * (es[j] * inv)
    out_ref[:, :D] = jnp.maximum(
        jnp.dot(x, xw_ref[...], preferred_element_type=jnp.float32), 0.0)
    out_ref[:, D:] = jnp.maximum(
        jnp.dot(agg, nw_ref[...], preferred_element_type=jnp.float32), 0.0)


def _split_nb(ref_a, ref_b):
    def get(i):
        if i < NB_A:
            return ref_a[pl.ds(i * B0, B0), :]
        return ref_b[pl.ds((i - NB_A) * B0, B0), :]
    return get


def _final_body(f0_ref, f1a_ref, f1b_ref, g1a_ref, g1b_ref,
                aw1_ref, xw1_ref, nw1_ref,
                aw2_ref, xw2_ref, nw2_ref, fcw_ref, fcb_ref, out_ref):
    x0 = f0_ref[...]
    aw1 = aw1_ref[...]
    xa = jnp.dot(x0, aw1, preferred_element_type=jnp.float32)
    agg0 = _softmax_weighted_sum(_split_nb(f1a_ref, f1b_ref), N1, xa, aw1)
    g0 = jnp.concatenate([
        jnp.maximum(jnp.dot(x0, xw1_ref[...],
                            preferred_element_type=jnp.float32), 0.0),
        jnp.maximum(jnp.dot(agg0, nw1_ref[...],
                            preferred_element_type=jnp.float32), 0.0)], axis=1)
    aw2 = aw2_ref[...]
    xa2 = jnp.dot(g0, aw2, preferred_element_type=jnp.float32)
    agg2 = _softmax_weighted_sum(_split_nb(g1a_ref, g1b_ref), N1, xa2, aw2)
    h0 = jnp.concatenate([
        jnp.maximum(jnp.dot(g0, xw2_ref[...],
                            preferred_element_type=jnp.float32), 0.0),
        jnp.maximum(jnp.dot(agg2, nw2_ref[...],
                            preferred_element_type=jnp.float32), 0.0)], axis=1)
    nrm = jnp.sqrt(jnp.sum(h0 * h0, axis=1, keepdims=True))
    hn = h0 / jnp.maximum(nrm, 1e-12)
    out_ref[...] = (jnp.dot(hn, fcw_ref[...],
                            preferred_element_type=jnp.float32)
                    + fcb_ref[...])


def _run_l1(f1x, f2x, nb, att_w1, fcx_w1, fcn_w1):
    return pl.pallas_call(
        _l1_body,
        grid=(nb,),
        in_specs=[
            pl.BlockSpec((B0, D), lambda kk: (kk, 0)),
            pl.BlockSpec((N2, B0, D), lambda kk: (0, kk, 0)),
            pl.BlockSpec((D, H), lambda kk: (0, 0)),
            pl.BlockSpec((D, D), lambda kk: (0, 0)),
            pl.BlockSpec((D, D), lambda kk: (0, 0)),
        ],
        out_specs=pl.BlockSpec((B0, 2 * D), lambda kk: (kk, 0)),
        out_shape=jax.ShapeDtypeStruct((nb * B0, 2 * D), jnp.float32),
    )(f1x, f2x, att_w1, fcx_w1, fcn_w1)


def kernel(ids, feats, adj, att_w1, fcx_w1, fcn_w1, att_w2, fcx_w2, fcn_w2,
           fc_w, fc_b):
    k = jax.random.key(42)
    perm1 = jax.random.permutation(jax.random.fold_in(k, 0), MAX_DEG)
    perm2 = jax.random.permutation(jax.random.fold_in(k, 1), MAX_DEG)
    cols1 = jnp.zeros((32,), jnp.int32).at[:N1].set(perm1[:N1].astype(jnp.int32))
    cols2 = jnp.broadcast_to(
        perm2[:N2].astype(jnp.int32)[:, None], (N2, L))

    ids = ids.astype(jnp.int32)
    adj = adj.astype(jnp.int32)
    ra, rb = NB_A * B0, NB_B * B0
    f0, f1a, f2a = _make_sc_gather(0, ra, True)(ids, adj, feats, cols1, cols2)
    f1b, f2b = _make_sc_gather(ra, rb, False)(ids, adj, feats, cols1, cols2)

    g1a = _run_l1(f1a, f2a, NB_A, att_w1, fcx_w1, fcn_w1)
    g1b = _run_l1(f1b, f2b, NB_B, att_w1, fcx_w1, fcn_w1)

    return pl.pallas_call(
        _final_body,
        compiler_params=pltpu.CompilerParams(vmem_limit_bytes=100 * 2**20),
        out_shape=jax.ShapeDtypeStruct((B0, fc_w.shape[1]), jnp.float32),
    )(f0, f1a, f1b, g1a, g1b, att_w1, fcx_w1, fcn_w1, att_w2, fcx_w2, fcn_w2,
      fc_w, fc_b.reshape(1, -1))


# speedup vs baseline: 1.0215x; 1.0127x over previous
import functools

import jax
import jax.numpy as jnp
from jax import lax
from jax.experimental import pallas as pl
from jax.experimental.pallas import tpu as pltpu
from jax.experimental.pallas import tpu_sc as plsc

N_NODES = 100000
MAX_DEG = 32
D = 128
H = 32
N1 = 25
N2 = 10
B0 = 1024
B1 = B0 * N1
NC = 2
NS = 16
NW = NC * NS
L = 16
PW = B1 // NW
CH = PW // 4
SEEDS_PW = B0 // NW
NB_A = 17
NB_B = N1 - NB_A

@functools.cache
def _make_sc_gather(p_base, rows, with_f0):
  mesh = plsc.VectorSubcoreMesh(core_axis_name="c", subcore_axis_name="s")
  pw = rows // NW
  ch = pw // 4
  assert pw % L == 0 and ch % 8 == 0 and ch * 4 == pw

  def body(ids_hbm, adj_hbm, feats_hbm, cols1_hbm, cols2_hbm, *rest):
    if with_f0:
        f0_hbm, f1_hbm, f2_hbm = rest[0], rest[1], rest[2]
        rest = rest[3:]
    else:
        f1_hbm, f2_hbm = rest[0], rest[1]
        rest = rest[2:]
    (ids_v, cols1_v, cols2_v, rowids_v, colsel_v, adjrows_v,
     ids1p_v, idx2a_v, idx2b_v, bufa, bufb,
     sema, semb, sem_adj, semwa, semwb) = rest
    wid = lax.axis_index("s") * NC + lax.axis_index("c")
    p0 = wid * pw
    pg = p_base + p0
    bufs, sems, semws = (bufa, bufb), (sema, semb), (semwa, semwb)
    idxbufs = (idx2a_v, idx2b_v)

    pltpu.sync_copy(ids_hbm, ids_v)
    pltpu.sync_copy(cols1_hbm, cols1_v)
    pltpu.sync_copy(cols2_hbm, cols2_v)

    def idx_body(c, carry):
        off = pl.multiple_of(c * L, L)
        p = pg + off + lax.iota(jnp.int32, L)
        i = lax.shift_right_logical(p, 10)
        b = lax.bitwise_and(p, B0 - 1)
        rowids_v[pl.ds(off, L)] = plsc.load_gather(ids_v, [b])
        colsel_v[pl.ds(off, L)] = plsc.load_gather(cols1_v, [i])
        return carry
    lax.fori_loop(0, pw // L, idx_body, 0)

    pltpu.async_copy(adj_hbm.at[rowids_v], adjrows_v, sem_adj).wait()

    def sel_body(c, carry):
        off = pl.multiple_of(c * L, L)
        lidx = off + lax.iota(jnp.int32, L)
        cs = colsel_v[pl.ds(off, L)]
        ids1p_v[pl.ds(off, L)] = plsc.load_gather(adjrows_v, [lidx, cs])
        return carry
    lax.fori_loop(0, pw // L, sel_body, 0)

    adj2 = pltpu.async_copy(adj_hbm.at[ids1p_v], adjrows_v, sem_adj)

    def _sel2(jcol, dst):
        def sbody(c, carry):
            off = pl.multiple_of(c * L, L)
            lidx = off + lax.iota(jnp.int32, L)
            dst[pl.ds(off, L)] = plsc.load_gather(adjrows_v, [lidx, jcol])
            return carry
        lax.fori_loop(0, pw // L, sbody, 0)

    chunks = []
    for h in range(4):
        chunks.append((ids1p_v.at[pl.ds(h * ch, ch)],
                       f1_hbm.at[pl.ds(p0 + h * ch, ch)],
                       True if h == 0 else None))
    for j in range(N2):
        for h in range(4):
            chunks.append((idxbufs[j & 1].at[pl.ds(h * ch, ch)],
                           f2_hbm.at[j, pl.ds(p0 + h * ch, ch)],
                           (j + 1, h) if (h == 0 and j + 1 < N2) else None))

    prev = None
    wbd = [None, None]
    for n, (idx_slice, dst, hook) in enumerate(chunks):
        bi = n & 1
        if wbd[bi] is not None:
            wbd[bi].wait()
            wbd[bi] = None
        d = pltpu.async_copy(feats_hbm.at[idx_slice], bufs[bi], sems[bi])
        if prev is not None:
            pd, pbi, pdst = prev
            pd.wait()
            wbd[pbi] = pltpu.async_copy(bufs[pbi], pdst, semws[pbi])
        if hook is True:
            adj2.wait()
            _sel2(cols2_v[0, :], idxbufs[0])
        elif hook is not None:
            jn = hook[0]
            _sel2(cols2_v[jn, :], idxbufs[jn & 1])
        prev = (d, bi, dst)

    pd, pbi, pdst = prev
    pd.wait()
    wbd[pbi] = pltpu.async_copy(bufs[pbi], pdst, semws[pbi])
    if with_f0:
        r0 = wid * SEEDS_PW
        bi = 1 - pbi
        if wbd[bi] is not None:
            wbd[bi].wait()
            wbd[bi] = None
        d0 = pltpu.async_copy(
            feats_hbm.at[ids_v.at[pl.ds(r0, SEEDS_PW)]],
            bufs[bi].at[pl.ds(0, SEEDS_PW)], sems[bi])
        d0.wait()
        pltpu.sync_copy(bufs[bi].at[pl.ds(0, SEEDS_PW)],
                        f0_hbm.at[pl.ds(r0, SEEDS_PW)])
    else:
        if wbd[1 - pbi] is not None:
            wbd[1 - pbi].wait()
    wbd[pbi].wait()

  out_type = []
  if with_f0:
      out_type.append(jax.ShapeDtypeStruct((B0, D), jnp.float32))
  out_type += [
      jax.ShapeDtypeStruct((rows, D), jnp.float32),
      jax.ShapeDtypeStruct((N2, rows, D), jnp.float32),
  ]
  return pl.kernel(
      body,
      mesh=mesh,
      compiler_params=pltpu.CompilerParams(
          needs_layout_passes=False, use_tc_tiling_on_sc=False),
      out_type=out_type,
      scratch_types=[
          pltpu.VMEM((B0,), jnp.int32),
          pltpu.VMEM((32,), jnp.int32),
          pltpu.VMEM((N2, L), jnp.int32),
          pltpu.VMEM((pw,), jnp.int32),
          pltpu.VMEM((pw,), jnp.int32),
          pltpu.VMEM((pw, MAX_DEG), jnp.int32),
          pltpu.VMEM((pw,), jnp.int32),
          pltpu.VMEM((pw,), jnp.int32),
          pltpu.VMEM((pw,), jnp.int32),
          pltpu.VMEM((ch, D), jnp.float32),
          pltpu.VMEM((ch, D), jnp.float32),
          pltpu.SemaphoreType.DMA,
          pltpu.SemaphoreType.DMA,
          pltpu.SemaphoreType.DMA,
          pltpu.SemaphoreType.DMA,
          pltpu.SemaphoreType.DMA,
      ],
  )


def _softmax_weighted_sum(get_nb, n, xa, aw):
    scores = []
    for j in range(n):
        na = jnp.dot(get_nb(j), aw, preferred_element_type=jnp.float32)
        scores.append(jnp.sum(na * xa, axis=1, keepdims=True))
    m = scores[0]
    for s in scores[1:]:
        m = jnp.maximum(m, s)
    es = [jnp.exp(s - m) for s in scores]
    z = es[0]
    for e in es[1:]:
        z = z + e
    inv = 1.0 / z
    agg = get_nb(0) * (es[0] * inv)
    for j in range(1, n):
        agg = agg + get_nb(j) * (es[j] * inv)
    return agg


def _l1_body(f1_ref, f2_ref, aw_ref, xw_ref, nw_ref, out_ref):
    x = f1_ref[...]
    aw = aw_ref[...]
    xa = jnp.dot(x, aw, preferred_element_type=jnp.float32)
    f2all = f2_ref[...].reshape(N2 * B0, D)
    na = jnp.dot(f2all, aw, preferred_element_type=jnp.float32)
    xa_rep = jnp.concatenate([xa] * N2, axis=0)
    s_all = jnp.sum(na * xa_rep, axis=1, keepdims=True)
    scores = [lax.slice(s_all, (j * B0, 0), ((j + 1) * B0, 1)) for j in range(N2)]
    m = scores[0]
    for s in scores[1:]:
        m = jnp.maximum(m, s)
    es = [jnp.exp(s - m) for s in scores]
    z = es[0]
    for e in es[1:]:
        z = z + e
    inv = 1.0 / z
    agg = f2_ref[0] * (es[0] * inv)
    for j in range(1, N2):
        agg = agg + f2_ref[j] * (es[j] * inv)
    out_ref[:, :D] = jnp.maximum(
        jnp.dot(x, xw_ref[...], preferred_element_type=jnp.float32), 0.0)
    out_ref[:, D:] = jnp.maximum(
        jnp.dot(agg, nw_ref[...], preferred_element_type=jnp.float32), 0.0)


def _split_nb(ref_a, ref_b):
    def get(i):
        if i < NB_A:
            return ref_a[pl.ds(i * B0, B0), :]
        return ref_b[pl.ds((i - NB_A) * B0, B0), :]
    return get


def _final_body(f0_ref, f1a_ref, f1b_ref, g1a_ref, g1b_ref,
                aw1_ref, xw1_ref, nw1_ref,
                aw2_ref, xw2_ref, nw2_ref, fcw_ref, fcb_ref, out_ref):
    x0 = f0_ref[...]
    aw1 = aw1_ref[...]
    xa = jnp.dot(x0, aw1, preferred_element_type=jnp.float32)
    agg0 = _softmax_weighted_sum(_split_nb(f1a_ref, f1b_ref), N1, xa, aw1)
    g0 = jnp.concatenate([
        jnp.maximum(jnp.dot(x0, xw1_ref[...],
                            preferred_element_type=jnp.float32), 0.0),
        jnp.maximum(jnp.dot(agg0, nw1_ref[...],
                            preferred_element_type=jnp.float32), 0.0)], axis=1)
    aw2 = aw2_ref[...]
    xa2 = jnp.dot(g0, aw2, preferred_element_type=jnp.float32)
    agg2 = _softmax_weighted_sum(_split_nb(g1a_ref, g1b_ref), N1, xa2, aw2)
    h0 = jnp.concatenate([
        jnp.maximum(jnp.dot(g0, xw2_ref[...],
                            preferred_element_type=jnp.float32), 0.0),
        jnp.maximum(jnp.dot(agg2, nw2_ref[...],
                            preferred_element_type=jnp.float32), 0.0)], axis=1)
    nrm = jnp.sqrt(jnp.sum(h0 * h0, axis=1, keepdims=True))
    hn = h0 / jnp.maximum(nrm, 1e-12)
    out_ref[...] = (jnp.dot(hn, fcw_ref[...],
                            preferred_element_type=jnp.float32)
                    + fcb_ref[...])


def _run_l1(f1x, f2x, nb, att_w1, fcx_w1, fcn_w1):
    return pl.pallas_call(
        _l1_body,
        grid=(nb,),
        in_specs=[
            pl.BlockSpec((B0, D), lambda kk: (kk, 0)),
            pl.BlockSpec((N2, B0, D), lambda kk: (0, kk, 0)),
            pl.BlockSpec((D, H), lambda kk: (0, 0)),
            pl.BlockSpec((D, D), lambda kk: (0, 0)),
            pl.BlockSpec((D, D), lambda kk: (0, 0)),
        ],
        out_specs=pl.BlockSpec((B0, 2 * D), lambda kk: (kk, 0)),
        out_shape=jax.ShapeDtypeStruct((nb * B0, 2 * D), jnp.float32),
    )(f1x, f2x, att_w1, fcx_w1, fcn_w1)


def kernel(ids, feats, adj, att_w1, fcx_w1, fcn_w1, att_w2, fcx_w2, fcn_w2,
           fc_w, fc_b):
    k = jax.random.key(42)
    perm1 = jax.random.permutation(jax.random.fold_in(k, 0), MAX_DEG)
    perm2 = jax.random.permutation(jax.random.fold_in(k, 1), MAX_DEG)
    cols1 = jnp.zeros((32,), jnp.int32).at[:N1].set(perm1[:N1].astype(jnp.int32))
    cols2 = jnp.broadcast_to(
        perm2[:N2].astype(jnp.int32)[:, None], (N2, L))

    ids = ids.astype(jnp.int32)
    adj = adj.astype(jnp.int32)
    ra, rb = NB_A * B0, NB_B * B0
    f0, f1a, f2a = _make_sc_gather(0, ra, True)(ids, adj, feats, cols1, cols2)
    f1b, f2b = _make_sc_gather(ra, rb, False)(ids, adj, feats, cols1, cols2)

    g1a = _run_l1(f1a, f2a, NB_A, att_w1, fcx_w1, fcn_w1)
    g1b = _run_l1(f1b, f2b, NB_B, att_w1, fcx_w1, fcn_w1)

    return pl.pallas_call(
        _final_body,
        compiler_params=pltpu.CompilerParams(vmem_limit_bytes=100 * 2**20),
        out_shape=jax.ShapeDtypeStruct((B0, fc_w.shape[1]), jnp.float32),
    )(f0, f1a, f1b, g1a, g1b, att_w1, fcx_w1, fcn_w1, att_w2, fcx_w2, fcn_w2,
      fc_w, fc_b.reshape(1, -1))
